# baseline (device time: 29096 ns/iter reference)
import jax
import jax.numpy as jnp
from jax import lax
from jax.experimental import pallas as pl
from jax.experimental.pallas import tpu as pltpu

N_DEV = 4
B = 2
S = 512
S_PER = 128
D = 512
HD = 256
DH = 64
NH = 4

BF = jnp.bfloat16
F32 = jnp.float32


def _body(x_ref, wq_ref, wk_ref, wv_ref, wo_ref, out_ref,
          xsend, xfull, qf, kf, vf, pout, rsbuf,
          ag_send_sems, ag_recv_sems, rs_send_sems, rs_recv_sems):
    me = lax.axis_index("i")

    barrier = pltpu.get_barrier_semaphore()
    for off in (1, 2, 3):
        peer = lax.rem(me + off, N_DEV)
        pl.semaphore_signal(barrier, inc=1, device_id=(peer,),
                            device_id_type=pl.DeviceIdType.MESH)
    pl.semaphore_wait(barrier, N_DEV - 1)

    for b in range(B):
        xsend[b] = x_ref[b].astype(BF)

    ag_rdmas = []
    for b in range(B):
        for off in (1, 2, 3):
            peer = lax.rem(me + off, N_DEV)
            rdma = pltpu.make_async_remote_copy(
                src_ref=xsend.at[b],
                dst_ref=xfull.at[b, me],
                send_sem=ag_send_sems.at[b * 3 + off - 1],
                recv_sem=ag_recv_sems.at[b, me],
                device_id=(peer,),
                device_id_type=pl.DeviceIdType.MESH,
            )
            rdma.start()
            ag_rdmas.append(rdma)

    for c in range(N_DEV):
        @pl.when(me == c)
        def _(c=c):
            for b in range(B):
                xfull[b, c] = xsend[b]

    wq = wq_ref[...].astype(BF)
    wk = wk_ref[...].astype(BF)
    wv = wv_ref[...].astype(BF)
    wo = wo_ref[...].astype(BF)

    pos = lax.broadcasted_iota(jnp.int32, (S, HD), 0).astype(F32)
    lane = lax.broadcasted_iota(jnp.int32, (S, HD), 1)
    d_in = lane % DH
    d_even = (d_in - (d_in % 2)).astype(F32)
    inv_freq = jnp.exp(d_even * (-jnp.log(10000.0) / DH))
    ang = pos * inv_freq
    cos_t = jnp.cos(ang)
    sin_t = jnp.sin(ang)
    even_mask = (d_in % 2) == 0

    def rot_rows(t, c):
        t_next = pltpu.roll(t, HD - 1, 1)
        t_prev = pltpu.roll(t, 1, 1)
        rows = slice(c * S_PER, (c + 1) * S_PER)
        t_r = jnp.where(even_mask[rows], -t_next, t_prev)
        return t * cos_t[rows] + t_r * sin_t[rows]

    rs_rdmas = []
    for b in range(B):
        for c in range(N_DEV):
            @pl.when(me != c)
            def _(c=c, b=b):
                pltpu.make_async_remote_copy(
                    src_ref=xsend.at[b],
                    dst_ref=xfull.at[b, c],
                    send_sem=ag_send_sems.at[0],
                    recv_sem=ag_recv_sems.at[b, c],
                    device_id=(c,),
                    device_id_type=pl.DeviceIdType.MESH,
                ).wait_recv()
            xc = xfull[b, c]
            rows = slice(c * S_PER, (c + 1) * S_PER)
            qc = jnp.dot(xc, wq, preferred_element_type=F32)
            kc = jnp.dot(xc, wk, preferred_element_type=F32)
            vc = jnp.dot(xc, wv, preferred_element_type=F32)
            qf[rows] = rot_rows(qc, c).astype(BF)
            kf[rows] = rot_rows(kc, c).astype(BF)
            vf[rows] = vc.astype(BF)

        q = qf[...]
        k = kf[...]
        v = vf[...]

        ctx_heads = []
        for h in range(NH):
            qh = q[:, h * DH:(h + 1) * DH]
            kh = k[:, h * DH:(h + 1) * DH]
            vh = v[:, h * DH:(h + 1) * DH]
            s = lax.dot_general(qh, kh, (((1,), (1,)), ((), ())),
                                preferred_element_type=F32) * 0.125
            s = s - jnp.max(s, axis=-1, keepdims=True)
            w = jnp.exp(s)
            w = (w / jnp.sum(w, axis=-1, keepdims=True)).astype(BF)
            ctx_heads.append(
                jnp.dot(w, vh, preferred_element_type=F32).astype(BF))
        ctx = jnp.concatenate(ctx_heads, axis=1)

        for c in range(N_DEV):
            obc = jnp.dot(ctx[c * S_PER:(c + 1) * S_PER], wo,
                          preferred_element_type=F32).astype(BF)
            pout[b, c] = obc

            @pl.when(me == c)
            def _(c=c, b=b):
                rsbuf[b, c] = pout[b, c]

            @pl.when(me != c)
            def _(c=c, b=b):
                rdma = pltpu.make_async_remote_copy(
                    src_ref=pout.at[b, c],
                    dst_ref=rsbuf.at[b, me],
                    send_sem=rs_send_sems.at[b, c],
                    recv_sem=rs_recv_sems.at[b, me],
                    device_id=(c,),
                    device_id_type=pl.DeviceIdType.MESH,
                )
                rdma.start()
            rs_rdmas.append((b, c))

    for b in range(B):
        for off in (1, 2, 3):
            src = lax.rem(me + off, N_DEV)
            pltpu.make_async_remote_copy(
                src_ref=pout.at[0, 0],
                dst_ref=rsbuf.at[b, src],
                send_sem=rs_send_sems.at[0, 0],
                recv_sem=rs_recv_sems.at[b, src],
                device_id=(src,),
                device_id_type=pl.DeviceIdType.MESH,
            ).wait_recv()
        out_ref[b] = (rsbuf[b, 0].astype(F32) + rsbuf[b, 1].astype(F32)
                      + rsbuf[b, 2].astype(F32) + rsbuf[b, 3].astype(F32))

    for rdma in ag_rdmas:
        rdma.wait_send()
    for b, c in rs_rdmas:
        @pl.when(me != c)
        def _(b=b, c=c):
            pltpu.make_async_remote_copy(
                src_ref=pout.at[b, c],
                dst_ref=rsbuf.at[b, me],
                send_sem=rs_send_sems.at[b, c],
                recv_sem=rs_recv_sems.at[b, me],
                device_id=(c,),
                device_id_type=pl.DeviceIdType.MESH,
            ).wait_send()


def kernel(x, Wq, Wk, Wv, Wo):
    return pl.pallas_call(
        _body,
        out_shape=jax.ShapeDtypeStruct((B, S_PER, D), jnp.float32),
        in_specs=[pl.BlockSpec(memory_space=pltpu.VMEM)] * 5,
        out_specs=pl.BlockSpec(memory_space=pltpu.VMEM),
        scratch_shapes=[
            pltpu.VMEM((B, S_PER, D), BF),
            pltpu.VMEM((B, N_DEV, S_PER, D), BF),
            pltpu.VMEM((S, HD), BF),
            pltpu.VMEM((S, HD), BF),
            pltpu.VMEM((S, HD), BF),
            pltpu.VMEM((B, N_DEV, S_PER, D), BF),
            pltpu.VMEM((B, N_DEV, S_PER, D), BF),
            pltpu.SemaphoreType.DMA((B * 3,)),
            pltpu.SemaphoreType.DMA((B, N_DEV)),
            pltpu.SemaphoreType.DMA((B, N_DEV)),
            pltpu.SemaphoreType.DMA((B, N_DEV)),
        ],
        compiler_params=pltpu.CompilerParams(collective_id=0),
    )(x, Wq, Wk, Wv, Wo)


# device time: 28501 ns/iter; 1.0209x vs baseline; 1.0209x over previous
import jax
import jax.numpy as jnp
from jax import lax
from jax.experimental import pallas as pl
from jax.experimental.pallas import tpu as pltpu

N_DEV = 4
B = 2
S = 512
S_PER = 128
D = 512
HD = 256
DH = 64
NH = 4

BF = jnp.bfloat16
F32 = jnp.float32


def _body(x_ref, wq_ref, wk_ref, wv_ref, wo_ref, out_ref,
          xsend, xfull, pout, rsbuf,
          ag_send_sems, ag_recv_sems, rs_send_sems, rs_recv_sems):
    me = lax.axis_index("i")

    barrier = pltpu.get_barrier_semaphore()
    for off in (1, 2, 3):
        peer = lax.rem(me + off, N_DEV)
        pl.semaphore_signal(barrier, inc=1, device_id=(peer,),
                            device_id_type=pl.DeviceIdType.MESH)
    pl.semaphore_wait(barrier, N_DEV - 1)

    for b in range(B):
        xsend[b] = x_ref[b].astype(BF)

    ag_rdmas = []
    for b in range(B):
        for off in (1, 2, 3):
            peer = lax.rem(me + off, N_DEV)
            rdma = pltpu.make_async_remote_copy(
                src_ref=xsend.at[b],
                dst_ref=xfull.at[b, me],
                send_sem=ag_send_sems.at[b * 3 + off - 1],
                recv_sem=ag_recv_sems.at[b, me],
                device_id=(peer,),
                device_id_type=pl.DeviceIdType.MESH,
            )
            rdma.start()
            ag_rdmas.append(rdma)

    for c in range(N_DEV):
        @pl.when(me == c)
        def _(c=c):
            for b in range(B):
                xfull[b, c] = xsend[b]

    wq = wq_ref[...].astype(BF)
    wk = wk_ref[...].astype(BF)
    wv = wv_ref[...].astype(BF)
    wo = wo_ref[...].astype(BF)

    pos = lax.broadcasted_iota(jnp.int32, (S, HD), 0).astype(F32)
    lane = lax.broadcasted_iota(jnp.int32, (S, HD), 1)
    d_in = lane % DH
    d_even = (d_in - (d_in % 2)).astype(F32)
    inv_freq = jnp.exp(d_even * (-jnp.log(10000.0) / DH))
    ang = pos * inv_freq
    cos_t = jnp.cos(ang)
    sin_t = jnp.sin(ang)
    even_mask = (d_in % 2) == 0

    def rot(t):
        t_next = pltpu.roll(t, HD - 1, 1)
        t_prev = pltpu.roll(t, 1, 1)
        t_r = jnp.where(even_mask, -t_next, t_prev)
        return t * cos_t + t_r * sin_t

    rs_sent = []
    for b in range(B):
        for off in (1, 2, 3):
            src = lax.rem(me + off, N_DEV)
            pltpu.make_async_remote_copy(
                src_ref=xsend.at[b],
                dst_ref=xfull.at[b, src],
                send_sem=ag_send_sems.at[0],
                recv_sem=ag_recv_sems.at[b, src],
                device_id=(src,),
                device_id_type=pl.DeviceIdType.MESH,
            ).wait_recv()

        xb = jnp.concatenate([xfull[b, c] for c in range(N_DEV)], axis=0)
        q = rot(jnp.dot(xb, wq, preferred_element_type=F32)).astype(BF)
        k = rot(jnp.dot(xb, wk, preferred_element_type=F32)).astype(BF)
        v = jnp.dot(xb, wv, preferred_element_type=F32).astype(BF)

        ctx_heads = []
        for h in range(NH):
            qh = q[:, h * DH:(h + 1) * DH]
            kh = k[:, h * DH:(h + 1) * DH]
            vh = v[:, h * DH:(h + 1) * DH]
            s = lax.dot_general(qh, kh, (((1,), (1,)), ((), ())),
                                preferred_element_type=F32) * 0.125
            s = s - jnp.max(s, axis=-1, keepdims=True)
            w = jnp.exp(s)
            w = (w / jnp.sum(w, axis=-1, keepdims=True)).astype(BF)
            ctx_heads.append(
                jnp.dot(w, vh, preferred_element_type=F32).astype(BF))
        ctx = jnp.concatenate(ctx_heads, axis=1)

        for c in range(N_DEV):
            obc = jnp.dot(ctx[c * S_PER:(c + 1) * S_PER], wo,
                          preferred_element_type=F32).astype(BF)
            pout[b, c] = obc

            @pl.when(me == c)
            def _(c=c, b=b):
                rsbuf[b, c] = pout[b, c]

            @pl.when(me != c)
            def _(c=c, b=b):
                pltpu.make_async_remote_copy(
                    src_ref=pout.at[b, c],
                    dst_ref=rsbuf.at[b, me],
                    send_sem=rs_send_sems.at[b, c],
                    recv_sem=rs_recv_sems.at[b, me],
                    device_id=(c,),
                    device_id_type=pl.DeviceIdType.MESH,
                ).start()
            rs_sent.append((b, c))

    for b in range(B):
        for off in (1, 2, 3):
            src = lax.rem(me + off, N_DEV)
            pltpu.make_async_remote_copy(
                src_ref=pout.at[0, 0],
                dst_ref=rsbuf.at[b, src],
                send_sem=rs_send_sems.at[0, 0],
                recv_sem=rs_recv_sems.at[b, src],
                device_id=(src,),
                device_id_type=pl.DeviceIdType.MESH,
            ).wait_recv()
        out_ref[b] = (rsbuf[b, 0].astype(F32) + rsbuf[b, 1].astype(F32)
                      + rsbuf[b, 2].astype(F32) + rsbuf[b, 3].astype(F32))

    for rdma in ag_rdmas:
        rdma.wait_send()
    for b, c in rs_sent:
        @pl.when(me != c)
        def _(b=b, c=c):
            pltpu.make_async_remote_copy(
                src_ref=pout.at[b, c],
                dst_ref=rsbuf.at[b, me],
                send_sem=rs_send_sems.at[b, c],
                recv_sem=rs_recv_sems.at[b, me],
                device_id=(c,),
                device_id_type=pl.DeviceIdType.MESH,
            ).wait_send()


def kernel(x, Wq, Wk, Wv, Wo):
    return pl.pallas_call(
        _body,
        out_shape=jax.ShapeDtypeStruct((B, S_PER, D), jnp.float32),
        in_specs=[pl.BlockSpec(memory_space=pltpu.VMEM)] * 5,
        out_specs=pl.BlockSpec(memory_space=pltpu.VMEM),
        scratch_shapes=[
            pltpu.VMEM((B, S_PER, D), BF),
            pltpu.VMEM((B, N_DEV, S_PER, D), BF),
            pltpu.VMEM((B, N_DEV, S_PER, D), BF),
            pltpu.VMEM((B, N_DEV, S_PER, D), BF),
            pltpu.SemaphoreType.DMA((B * 3,)),
            pltpu.SemaphoreType.DMA((B, N_DEV)),
            pltpu.SemaphoreType.DMA((B, N_DEV)),
            pltpu.SemaphoreType.DMA((B, N_DEV)),
        ],
        compiler_params=pltpu.CompilerParams(collective_id=0),
    )(x, Wq, Wk, Wv, Wo)


# device time: 28228 ns/iter; 1.0307x vs baseline; 1.0097x over previous
import jax
import jax.numpy as jnp
from jax import lax
from jax.experimental import pallas as pl
from jax.experimental.pallas import tpu as pltpu

N_DEV = 4
B = 2
S = 512
S_PER = 128
D = 512
HD = 256
DH = 64
NH = 4

BF = jnp.bfloat16
F32 = jnp.float32


def _body(x_ref, wq_ref, wk_ref, wv_ref, wo_ref, out_ref,
          xsend, xfull, qf, kf, vf, pout, rsbuf,
          ag_send_sems, ag_recv_sems, rs_send_sems, rs_recv_sems):
    me = lax.axis_index("i")

    barrier = pltpu.get_barrier_semaphore()
    for off in (1, 2, 3):
        peer = lax.rem(me + off, N_DEV)
        pl.semaphore_signal(barrier, inc=1, device_id=(peer,),
                            device_id_type=pl.DeviceIdType.MESH)
    pl.semaphore_wait(barrier, N_DEV - 1)

    for b in range(B):
        xsend[b] = x_ref[b].astype(BF)

    ag_rdmas = []
    for b in range(B):
        for off in (1, 2, 3):
            peer = lax.rem(me + off, N_DEV)
            rdma = pltpu.make_async_remote_copy(
                src_ref=xsend.at[b],
                dst_ref=xfull.at[b, pl.ds(me * S_PER, S_PER)],
                send_sem=ag_send_sems.at[b * 3 + off - 1],
                recv_sem=ag_recv_sems.at[b, me],
                device_id=(peer,),
                device_id_type=pl.DeviceIdType.MESH,
            )
            rdma.start()
            ag_rdmas.append(rdma)

    xfull[0, pl.ds(me * S_PER, S_PER)] = xsend[0]
    xfull[1, pl.ds(me * S_PER, S_PER)] = xsend[1]

    wq = wq_ref[...].astype(BF)
    wk = wk_ref[...].astype(BF)
    wv = wv_ref[...].astype(BF)
    wo = wo_ref[...].astype(BF)

    pos = lax.broadcasted_iota(jnp.int32, (S, HD), 0).astype(F32)
    lane = lax.broadcasted_iota(jnp.int32, (S, HD), 1)
    d_in = lane % DH
    d_even = (d_in - (d_in % 2)).astype(F32)
    inv_freq = jnp.exp(d_even * (-jnp.log(10000.0) / DH))
    ang = pos * inv_freq
    cos_t = jnp.cos(ang)
    sin_t = jnp.sin(ang)
    even_mask = (d_in % 2) == 0

    def rot(t):
        t_next = pltpu.roll(t, HD - 1, 1)
        t_prev = pltpu.roll(t, 1, 1)
        t_r = jnp.where(even_mask, -t_next, t_prev)
        return t * cos_t + t_r * sin_t

    for b in range(B):
        for off in (1, 2, 3):
            src = lax.rem(me + off, N_DEV)
            pltpu.make_async_remote_copy(
                src_ref=xsend.at[b],
                dst_ref=xfull.at[b, pl.ds(src * S_PER, S_PER)],
                send_sem=ag_send_sems.at[0],
                recv_sem=ag_recv_sems.at[b, src],
                device_id=(src,),
                device_id_type=pl.DeviceIdType.MESH,
            ).wait_recv()

        xb = xfull[b]
        qf[...] = rot(jnp.dot(xb, wq, preferred_element_type=F32)).astype(BF)
        kf[...] = rot(jnp.dot(xb, wk, preferred_element_type=F32)).astype(BF)
        vf[...] = jnp.dot(xb, wv, preferred_element_type=F32).astype(BF)

        for j in (1, 2, 3, 0):
            block = lax.rem(me + j, N_DEV)
            qblk = qf[pl.ds(block * S_PER, S_PER), :]
            ctx_heads = []
            for h in range(NH):
                qh = qblk[:, h * DH:(h + 1) * DH]
                kh = kf[:, h * DH:(h + 1) * DH]
                vh = vf[:, h * DH:(h + 1) * DH]
                s = lax.dot_general(qh, kh, (((1,), (1,)), ((), ())),
                                    preferred_element_type=F32) * 0.125
                w = jnp.exp(s)
                denom = jnp.sum(w, axis=-1, keepdims=True)
                ctx_h = jnp.dot(w.astype(BF), vh,
                                preferred_element_type=F32) / denom
                ctx_heads.append(ctx_h.astype(BF))
            ctx = jnp.concatenate(ctx_heads, axis=1)
            ob = jnp.dot(ctx, wo, preferred_element_type=F32).astype(BF)

            if j == 0:
                rsbuf[b, pl.ds(me, 1)] = ob[None]
            else:
                pout[b, pl.ds(block, 1)] = ob[None]
                pltpu.make_async_remote_copy(
                    src_ref=pout.at[b, block],
                    dst_ref=rsbuf.at[b, me],
                    send_sem=rs_send_sems.at[b, j - 1],
                    recv_sem=rs_recv_sems.at[b, me],
                    device_id=(block,),
                    device_id_type=pl.DeviceIdType.MESH,
                ).start()

    for b in range(B):
        for off in (1, 2, 3):
            src = lax.rem(me + off, N_DEV)
            pltpu.make_async_remote_copy(
                src_ref=pout.at[0, 0],
                dst_ref=rsbuf.at[b, src],
                send_sem=rs_send_sems.at[0, 0],
                recv_sem=rs_recv_sems.at[b, src],
                device_id=(src,),
                device_id_type=pl.DeviceIdType.MESH,
            ).wait_recv()
        out_ref[b] = (rsbuf[b, 0].astype(F32) + rsbuf[b, 1].astype(F32)
                      + rsbuf[b, 2].astype(F32) + rsbuf[b, 3].astype(F32))

    for rdma in ag_rdmas:
        rdma.wait_send()
    for b in range(B):
        for j in (1, 2, 3):
            block = lax.rem(me + j, N_DEV)
            pltpu.make_async_remote_copy(
                src_ref=pout.at[b, block],
                dst_ref=rsbuf.at[b, me],
                send_sem=rs_send_sems.at[b, j - 1],
                recv_sem=rs_recv_sems.at[b, me],
                device_id=(block,),
                device_id_type=pl.DeviceIdType.MESH,
            ).wait_send()


def kernel(x, Wq, Wk, Wv, Wo):
    return pl.pallas_call(
        _body,
        out_shape=jax.ShapeDtypeStruct((B, S_PER, D), jnp.float32),
        in_specs=[pl.BlockSpec(memory_space=pltpu.VMEM)] * 5,
        out_specs=pl.BlockSpec(memory_space=pltpu.VMEM),
        scratch_shapes=[
            pltpu.VMEM((B, S_PER, D), BF),
            pltpu.VMEM((B, S, D), BF),
            pltpu.VMEM((S, HD), BF),
            pltpu.VMEM((S, HD), BF),
            pltpu.VMEM((S, HD), BF),
            pltpu.VMEM((B, N_DEV, S_PER, D), BF),
            pltpu.VMEM((B, N_DEV, S_PER, D), BF),
            pltpu.SemaphoreType.DMA((B * 3,)),
            pltpu.SemaphoreType.DMA((B, N_DEV)),
            pltpu.SemaphoreType.DMA((B, 3)),
            pltpu.SemaphoreType.DMA((B, N_DEV)),
        ],
        compiler_params=pltpu.CompilerParams(collective_id=0),
    )(x, Wq, Wk, Wv, Wo)


# device time: 24998 ns/iter; 1.1639x vs baseline; 1.1292x over previous
import jax
import jax.numpy as jnp
from jax import lax
from jax.experimental import pallas as pl
from jax.experimental.pallas import tpu as pltpu

N_DEV = 4
B = 2
S = 512
S_PER = 128
D = 512
HD = 256
DH = 64
NH = 4

BF = jnp.bfloat16
F32 = jnp.float32


def _body(x_ref, wq_ref, wk_ref, wv_ref, wo_ref, out_ref,
          xsend, xfull, qf, kf, vf, pout, rsbuf,
          ag_send_sems, ag_recv_sems, rs_send_sems, rs_recv_sems):
    me = lax.axis_index("i")

    barrier = pltpu.get_barrier_semaphore()
    for off in (1, 2, 3):
        peer = lax.rem(me + off, N_DEV)
        pl.semaphore_signal(barrier, inc=1, device_id=(peer,),
                            device_id_type=pl.DeviceIdType.MESH)
    pl.semaphore_wait(barrier, N_DEV - 1)

    for b in range(B):
        xsend[b] = x_ref[b].astype(BF)

    ag_rdmas = []
    for b in range(B):
        for off in (1, 2, 3):
            peer = lax.rem(me + off, N_DEV)
            rdma = pltpu.make_async_remote_copy(
                src_ref=xsend.at[b],
                dst_ref=xfull.at[b, pl.ds(me * S_PER, S_PER)],
                send_sem=ag_send_sems.at[b * 3 + off - 1],
                recv_sem=ag_recv_sems.at[b, me],
                device_id=(peer,),
                device_id_type=pl.DeviceIdType.MESH,
            )
            rdma.start()
            ag_rdmas.append(rdma)

    xfull[0, pl.ds(me * S_PER, S_PER)] = xsend[0]
    xfull[1, pl.ds(me * S_PER, S_PER)] = xsend[1]

    wq = wq_ref[...].astype(BF)
    wk = wk_ref[...].astype(BF)
    wv = wv_ref[...].astype(BF)
    wo = wo_ref[...].astype(BF)

    pos = lax.broadcasted_iota(jnp.int32, (S, HD), 0).astype(F32)
    lane = lax.broadcasted_iota(jnp.int32, (S, HD), 1)
    d_in = lane % DH
    d_even = (d_in - (d_in % 2)).astype(F32)
    inv_freq = jnp.exp(d_even * (-jnp.log(10000.0) / DH))
    ang = pos * inv_freq
    cos_t = jnp.cos(ang)
    sin_t = jnp.sin(ang)
    even_mask = (d_in % 2) == 0

    def rot(t):
        t_next = pltpu.roll(t, HD - 1, 1)
        t_prev = pltpu.roll(t, 1, 1)
        t_r = jnp.where(even_mask, -t_next, t_prev)
        return t * cos_t + t_r * sin_t

    for b in range(B):
        for off in (1, 2, 3):
            src = lax.rem(me + off, N_DEV)
            pltpu.make_async_remote_copy(
                src_ref=xsend.at[b],
                dst_ref=xfull.at[b, pl.ds(src * S_PER, S_PER)],
                send_sem=ag_send_sems.at[0],
                recv_sem=ag_recv_sems.at[b, src],
                device_id=(src,),
                device_id_type=pl.DeviceIdType.MESH,
            ).wait_recv()

        xb = xfull[b]
        qf[...] = (rot(jnp.dot(xb, wq, preferred_element_type=F32))
                   * 0.125).astype(BF)
        kf[...] = rot(jnp.dot(xb, wk, preferred_element_type=F32)).astype(BF)
        vf[...] = jnp.dot(xb, wv, preferred_element_type=F32).astype(BF)

        ctx_heads = []
        for h in range(NH):
            qh = qf[:, h * DH:(h + 1) * DH]
            kh = kf[:, h * DH:(h + 1) * DH]
            vh = vf[:, h * DH:(h + 1) * DH]
            s = lax.dot_general(qh, kh, (((1,), (1,)), ((), ())),
                                preferred_element_type=F32)
            w = jnp.exp(s.astype(BF))
            denom = jnp.sum(w, axis=-1, keepdims=True, dtype=F32)
            ctx_h = jnp.dot(w, vh, preferred_element_type=F32) / denom
            ctx_heads.append(ctx_h.astype(BF))
        ctx = jnp.concatenate(ctx_heads, axis=1)
        for c in range(N_DEV):
            ob = jnp.dot(ctx[c * S_PER:(c + 1) * S_PER], wo,
                         preferred_element_type=F32).astype(BF)
            pout[b, c] = ob

            @pl.when(me == c)
            def _(c=c, b=b):
                rsbuf[b, c] = pout[b, c]

            @pl.when(me != c)
            def _(c=c, b=b):
                pltpu.make_async_remote_copy(
                    src_ref=pout.at[b, c],
                    dst_ref=rsbuf.at[b, me],
                    send_sem=rs_send_sems.at[b, c],
                    recv_sem=rs_recv_sems.at[b, me],
                    device_id=(c,),
                    device_id_type=pl.DeviceIdType.MESH,
                ).start()

    for b in range(B):
        for off in (1, 2, 3):
            src = lax.rem(me + off, N_DEV)
            pltpu.make_async_remote_copy(
                src_ref=pout.at[0, 0],
                dst_ref=rsbuf.at[b, src],
                send_sem=rs_send_sems.at[0, 0],
                recv_sem=rs_recv_sems.at[b, src],
                device_id=(src,),
                device_id_type=pl.DeviceIdType.MESH,
            ).wait_recv()
        out_ref[b] = (rsbuf[b, 0].astype(F32) + rsbuf[b, 1].astype(F32)
                      + rsbuf[b, 2].astype(F32) + rsbuf[b, 3].astype(F32))

    for rdma in ag_rdmas:
        rdma.wait_send()
    for b in range(B):
        for c in range(N_DEV):
            @pl.when(me != c)
            def _(b=b, c=c):
                pltpu.make_async_remote_copy(
                    src_ref=pout.at[b, c],
                    dst_ref=rsbuf.at[b, me],
                    send_sem=rs_send_sems.at[b, c],
                    recv_sem=rs_recv_sems.at[b, me],
                    device_id=(c,),
                    device_id_type=pl.DeviceIdType.MESH,
                ).wait_send()


def kernel(x, Wq, Wk, Wv, Wo):
    return pl.pallas_call(
        _body,
        out_shape=jax.ShapeDtypeStruct((B, S_PER, D), jnp.float32),
        in_specs=[pl.BlockSpec(memory_space=pltpu.VMEM)] * 5,
        out_specs=pl.BlockSpec(memory_space=pltpu.VMEM),
        scratch_shapes=[
            pltpu.VMEM((B, S_PER, D), BF),
            pltpu.VMEM((B, S, D), BF),
            pltpu.VMEM((S, HD), BF),
            pltpu.VMEM((S, HD), BF),
            pltpu.VMEM((S, HD), BF),
            pltpu.VMEM((B, N_DEV, S_PER, D), BF),
            pltpu.VMEM((B, N_DEV, S_PER, D), BF),
            pltpu.SemaphoreType.DMA((B * 3,)),
            pltpu.SemaphoreType.DMA((B, N_DEV)),
            pltpu.SemaphoreType.DMA((B, N_DEV)),
            pltpu.SemaphoreType.DMA((B, N_DEV)),
        ],
        compiler_params=pltpu.CompilerParams(collective_id=0),
    )(x, Wq, Wk, Wv, Wo)
